# baseline (speedup 1.0000x reference)
import jax
import jax.numpy as jnp
from jax.experimental import pallas as pl
from jax.experimental.pallas import tpu as pltpu

_MARGIN = 0.2
_PAD_DIST = 100.0
_BIG = 1e9


def _make_body(bb, p, d):
    n = bb * p

    def body(np1_ref, np2_ref, e1_ref, e2_ref, gt_ref, out_ref, acc_ref):
        i = pl.program_id(0)

        @pl.when(i == 0)
        def _init():
            acc_ref[0] = 0.0
            acc_ref[1] = 0.0

        e1 = e1_ref[...]
        e2 = e2_ref[...]
        gtb = gt_ref[...] != 0
        a2 = jnp.sum(e1 * e1, axis=1, keepdims=True)
        ones = jnp.ones((1, d), dtype=jnp.float32)
        b2 = jax.lax.dot_general(ones, e2 * e2,
                                 (((1,), (1,)), ((), ())),
                                 preferred_element_type=jnp.float32)
        ab = jax.lax.dot_general(e1, e2, (((1,), (1,)), ((), ())),
                                 preferred_element_type=jnp.float32)
        d2full = a2 + b2 - 2.0 * ab

        row_i = jax.lax.broadcasted_iota(jnp.int32, (n, 1), 0)
        rowblk = row_i // p
        d2blk = jnp.zeros((n, p), dtype=jnp.float32)
        for k in range(bb):
            d2blk = jnp.where(rowblk == k, d2full[:, k * p:(k + 1) * p], d2blk)
        dist = jnp.sqrt(jnp.maximum(d2blk, 0.0))

        col_i = jax.lax.broadcasted_iota(jnp.int32, (1, p), 1)
        m1 = (row_i % p) < np1_ref[0]
        m2 = col_i < np2_ref[0]
        df = jnp.where(m1 & m2, dist, _PAD_DIST)
        mn = jnp.min(jnp.where(gtb, _BIG, df), axis=1, keepdims=True)
        val = df + _MARGIN - mn
        sel = gtb & (val > 0.0)
        acc_ref[0] += jnp.sum(jnp.where(sel, val, 0.0))
        acc_ref[1] += jnp.sum(sel.astype(jnp.float32))

        @pl.when(i == pl.num_programs(0) - 1)
        def _fin():
            total = acc_ref[0]
            cnt = acc_ref[1]
            out_ref[0, 0] = jnp.where(cnt > 0.0,
                                      total / jnp.maximum(cnt, 1.0), _MARGIN)

    return body


def kernel(embeddings1, embeddings2, gt_corr_ms, numPlanes1, numPlanes2,
           loss_weight):
    B, P, D = embeddings1.shape
    BB = 32
    N = BB * P
    nblk = B // BB
    e1f = embeddings1.reshape(B * P, D)
    e2f = embeddings2.reshape(B * P, D)
    gtf = gt_corr_ms.reshape(B * P, P).astype(jnp.int8)
    np1_rep = jnp.repeat(numPlanes1.astype(jnp.int32), P).reshape(nblk, N, 1)
    np2_rep = jnp.repeat(numPlanes2.astype(jnp.int32), P).reshape(nblk, N, 1)
    out = pl.pallas_call(
        _make_body(BB, P, D),
        grid=(nblk,),
        in_specs=[
            pl.BlockSpec((1, N, 1), lambda i: (i, 0, 0)),
            pl.BlockSpec((1, N, 1), lambda i: (i, 0, 0)),
            pl.BlockSpec((N, D), lambda i: (i, 0)),
            pl.BlockSpec((N, D), lambda i: (i, 0)),
            pl.BlockSpec((N, P), lambda i: (i, 0)),
        ],
        out_specs=pl.BlockSpec((1, 1), lambda i: (0, 0),
                               memory_space=pltpu.SMEM),
        out_shape=jax.ShapeDtypeStruct((1, 1), jnp.float32),
        scratch_shapes=[pltpu.SMEM((2,), jnp.float32)],
    )(np1_rep, np2_rep, e1f, e2f, gtf)
    return (loss_weight * out[0, 0]).astype(jnp.float32)


# speedup vs baseline: 3.9903x; 3.9903x over previous
import jax
import jax.numpy as jnp
from jax.experimental import pallas as pl
from jax.experimental.pallas import tpu as pltpu

_MARGIN = 0.2
_PAD_DIST = 100.0
_BIG = 1e9


def _make_body(bb, p, d):

    def body(code_ref, e1_ref, e2_ref, out_ref, acc_ref):
        i = pl.program_id(0)

        @pl.when(i == 0)
        def _init():
            acc_ref[0] = 0.0
            acc_ref[1] = 0.0

        e1 = e1_ref[...]
        e2 = e2_ref[...]
        code = code_ref[...]
        onescol = jnp.ones((bb, p, 1), dtype=jnp.float32)
        a2 = jnp.sum(e1 * e1, axis=2, keepdims=True)
        b2 = jnp.sum(e2 * e2, axis=2, keepdims=True)
        u = jnp.concatenate([-2.0 * e1, a2, onescol], axis=2)
        v = jnp.concatenate([e2, onescol, b2], axis=2)
        d2 = jax.lax.dot_general(u, v, (((2,), (2,)), ((0,), (0,))),
                                 preferred_element_type=jnp.float32)
        dist = jnp.sqrt(jnp.maximum(d2, 0.0))
        codei = code.astype(jnp.int32)
        validm = (codei & 1) != 0
        gtb = (codei & 2) != 0
        df = jnp.where(validm, dist, _PAD_DIST)
        mn = jnp.min(jnp.where(gtb, _BIG, df), axis=2, keepdims=True)
        val = df + _MARGIN - mn
        sel = gtb & (val > 0.0)
        acc_ref[0] += jnp.sum(jnp.where(sel, val, 0.0))
        acc_ref[1] += jnp.sum(sel.astype(jnp.float32))

        @pl.when(i == pl.num_programs(0) - 1)
        def _fin():
            total = acc_ref[0]
            cnt = acc_ref[1]
            out_ref[0, 0] = jnp.where(cnt > 0.0,
                                      total / jnp.maximum(cnt, 1.0), _MARGIN)

    return body


def kernel(embeddings1, embeddings2, gt_corr_ms, numPlanes1, numPlanes2,
           loss_weight):
    B, P, D = embeddings1.shape
    BB = 32
    nblk = B // BB
    r = jnp.arange(P)
    m1 = r[None, :] < numPlanes1[:, None]
    m2 = r[None, :] < numPlanes2[:, None]
    validm = m1[:, :, None] & m2[:, None, :]
    code = validm.astype(jnp.int8) + 2 * gt_corr_ms.astype(jnp.int8)
    out = pl.pallas_call(
        _make_body(BB, P, D),
        grid=(nblk,),
        in_specs=[
            pl.BlockSpec((BB, P, P), lambda i: (i, 0, 0)),
            pl.BlockSpec((BB, P, D), lambda i: (i, 0, 0)),
            pl.BlockSpec((BB, P, D), lambda i: (i, 0, 0)),
        ],
        out_specs=pl.BlockSpec((1, 1), lambda i: (0, 0),
                               memory_space=pltpu.SMEM),
        out_shape=jax.ShapeDtypeStruct((1, 1), jnp.float32),
        scratch_shapes=[pltpu.SMEM((2,), jnp.float32)],
    )(code, embeddings1, embeddings2)
    return (loss_weight * out[0, 0]).astype(jnp.float32)


# speedup vs baseline: 4.4514x; 1.1156x over previous
import jax
import jax.numpy as jnp
from jax.experimental import pallas as pl
from jax.experimental.pallas import tpu as pltpu

_MARGIN = 0.2
_PAD_DIST = 100.0
_BIG = 1e9


def _make_body(bb, p, d):

    def body(code_ref, e1_ref, e2_ref, out_ref, acc_ref):
        i = pl.program_id(0)

        @pl.when(i == 0)
        def _init():
            acc_ref[0] = 0.0
            acc_ref[1] = 0.0

        e1 = e1_ref[...]
        e2 = e2_ref[...]
        code = code_ref[...]
        onescol = jnp.ones((bb, p, 1), dtype=jnp.float32)
        a2 = jnp.sum(e1 * e1, axis=2, keepdims=True)
        b2 = jnp.sum(e2 * e2, axis=2, keepdims=True)
        u = jnp.concatenate([-2.0 * e1, a2, onescol], axis=2)
        v = jnp.concatenate([e2, onescol, b2], axis=2)
        d2 = jax.lax.dot_general(u, v, (((2,), (2,)), ((0,), (0,))),
                                 preferred_element_type=jnp.float32)
        dist = jnp.sqrt(jnp.maximum(d2, 0.0))
        codei = code.astype(jnp.int32)
        validm = (codei & 1) != 0
        gtb = (codei & 2) != 0
        df = jnp.where(validm, dist, _PAD_DIST)
        mn = jnp.min(jnp.where(gtb, _BIG, df), axis=2, keepdims=True)
        val = df + _MARGIN - mn
        sel = gtb & (val > 0.0)
        acc_ref[0] += jnp.sum(jnp.where(sel, val, 0.0))
        acc_ref[1] += jnp.sum(sel.astype(jnp.float32))

        @pl.when(i == pl.num_programs(0) - 1)
        def _fin():
            total = acc_ref[0]
            cnt = acc_ref[1]
            out_ref[0, 0] = jnp.where(cnt > 0.0,
                                      total / jnp.maximum(cnt, 1.0), _MARGIN)

    return body


def kernel(embeddings1, embeddings2, gt_corr_ms, numPlanes1, numPlanes2,
           loss_weight):
    B, P, D = embeddings1.shape
    BB = 64
    nblk = B // BB
    r = jnp.arange(P)
    m1 = r[None, :] < numPlanes1[:, None]
    m2 = r[None, :] < numPlanes2[:, None]
    validm = m1[:, :, None] & m2[:, None, :]
    code = validm.astype(jnp.int8) + 2 * gt_corr_ms.astype(jnp.int8)
    out = pl.pallas_call(
        _make_body(BB, P, D),
        grid=(nblk,),
        in_specs=[
            pl.BlockSpec((BB, P, P), lambda i: (i, 0, 0)),
            pl.BlockSpec((BB, P, D), lambda i: (i, 0, 0)),
            pl.BlockSpec((BB, P, D), lambda i: (i, 0, 0)),
        ],
        out_specs=pl.BlockSpec((1, 1), lambda i: (0, 0),
                               memory_space=pltpu.SMEM),
        out_shape=jax.ShapeDtypeStruct((1, 1), jnp.float32),
        scratch_shapes=[pltpu.SMEM((2,), jnp.float32)],
    )(code, embeddings1, embeddings2)
    return (loss_weight * out[0, 0]).astype(jnp.float32)


# speedup vs baseline: 4.5054x; 1.0121x over previous
import jax
import jax.numpy as jnp
from jax.experimental import pallas as pl
from jax.experimental.pallas import tpu as pltpu

_MARGIN = 0.2
_PAD_DIST = 100.0
_BIG = 1e9


def _make_body(bb, p, d):

    def body(code_ref, e1_ref, e2_ref, out_ref, acc_ref):
        i = pl.program_id(0)

        @pl.when(i == 0)
        def _init():
            acc_ref[0] = 0.0
            acc_ref[1] = 0.0

        e1 = e1_ref[...]
        e2 = e2_ref[...]
        code = code_ref[...]
        onescol = jnp.ones((bb, p, 1), dtype=jnp.float32)
        a2 = jnp.sum(e1 * e1, axis=2, keepdims=True)
        b2 = jnp.sum(e2 * e2, axis=2, keepdims=True)
        u = jnp.concatenate([-2.0 * e1, a2, onescol], axis=2)
        v = jnp.concatenate([e2, onescol, b2], axis=2)
        d2 = jax.lax.dot_general(u, v, (((2,), (2,)), ((0,), (0,))),
                                 preferred_element_type=jnp.float32)
        dist = jnp.sqrt(jnp.maximum(d2, 0.0))
        codei = code.astype(jnp.int32)
        validm = (codei & 1) != 0
        gtb = (codei & 2) != 0
        df = jnp.where(validm, dist, _PAD_DIST)
        mn = jnp.min(jnp.where(gtb, _BIG, df), axis=2, keepdims=True)
        val = df + _MARGIN - mn
        sel = gtb & (val > 0.0)
        acc_ref[0] += jnp.sum(jnp.where(sel, val, 0.0))
        acc_ref[1] += jnp.sum(sel.astype(jnp.float32))

        @pl.when(i == pl.num_programs(0) - 1)
        def _fin():
            total = acc_ref[0]
            cnt = acc_ref[1]
            out_ref[0, 0] = jnp.where(cnt > 0.0,
                                      total / jnp.maximum(cnt, 1.0), _MARGIN)

    return body


def kernel(embeddings1, embeddings2, gt_corr_ms, numPlanes1, numPlanes2,
           loss_weight):
    B, P, D = embeddings1.shape
    BB = 128
    nblk = B // BB
    r = jnp.arange(P)
    m1 = r[None, :] < numPlanes1[:, None]
    m2 = r[None, :] < numPlanes2[:, None]
    validm = m1[:, :, None] & m2[:, None, :]
    code = validm.astype(jnp.int8) + 2 * gt_corr_ms.astype(jnp.int8)
    out = pl.pallas_call(
        _make_body(BB, P, D),
        grid=(nblk,),
        in_specs=[
            pl.BlockSpec((BB, P, P), lambda i: (i, 0, 0)),
            pl.BlockSpec((BB, P, D), lambda i: (i, 0, 0)),
            pl.BlockSpec((BB, P, D), lambda i: (i, 0, 0)),
        ],
        out_specs=pl.BlockSpec((1, 1), lambda i: (0, 0),
                               memory_space=pltpu.SMEM),
        out_shape=jax.ShapeDtypeStruct((1, 1), jnp.float32),
        scratch_shapes=[pltpu.SMEM((2,), jnp.float32)],
    )(code, embeddings1, embeddings2)
    return (loss_weight * out[0, 0]).astype(jnp.float32)
